# baseline (speedup 1.0000x reference)
import functools

import jax
import jax.numpy as jnp
from jax import lax
from jax.experimental import pallas as pl
from jax.experimental.pallas import tpu as pltpu
from jax.experimental.pallas import tpu_sc as plsc

B = 16384
D = 32
NC = 2
NS = 16
L = 16
NW = NC * NS
BPW = B // NW
CHUNK = 128
NCH = BPW // CHUNK

BC = 65536


def _mv_body(x_ref, w_ref, b_ref, o_hbm, scr, sem):
    i = pl.program_id(0)
    scr[...] = jnp.sum(x_ref[...] * w_ref[...], axis=0) + b_ref[0, 0]
    cp = pltpu.make_async_copy(scr, o_hbm.at[pl.ds(i * BC, BC)], sem)
    cp.start()
    cp.wait()


def _matvec(table_t, w, bias):
    n = table_t.shape[1]
    grid = pl.cdiv(n, BC)
    return pl.pallas_call(
        _mv_body,
        grid=(grid,),
        in_specs=[
            pl.BlockSpec((D, BC), lambda i: (0, i)),
            pl.BlockSpec((D, 1), lambda i: (0, 0)),
            pl.BlockSpec((1, 1), lambda i: (0, 0)),
        ],
        out_specs=pl.BlockSpec(memory_space=pltpu.MemorySpace.HBM),
        out_shape=jax.ShapeDtypeStruct((grid * BC,), jnp.float32),
        scratch_shapes=[
            pltpu.VMEM((BC,), jnp.float32),
            pltpu.SemaphoreType.DMA,
        ],
    )(table_t, w, bias)


_mesh = plsc.VectorSubcoreMesh(core_axis_name="c", subcore_axis_name="s")


@functools.partial(
    pl.kernel,
    out_type=jax.ShapeDtypeStruct((B,), jnp.float32),
    mesh=_mesh,
    compiler_params=pltpu.CompilerParams(
        needs_layout_passes=False, use_tc_tiling_on_sc=False),
    scratch_types=[
        pltpu.VMEM((NCH, CHUNK), jnp.int32),
        pltpu.VMEM((BPW,), jnp.float32),
        pltpu.SemaphoreType.DMA,
    ],
)
def _sc_gather_movie(movies_hbm, sm_hbm, out_hbm, midx_v, sm_v, sem):
    wid = lax.axis_index("s") * NC + lax.axis_index("c")
    pltpu.sync_copy(movies_hbm.at[pl.ds(wid * NCH, NCH)], midx_v)
    copies = [
        pltpu.async_copy(
            sm_hbm.at[midx_v.at[j]], sm_v.at[pl.ds(j * CHUNK, CHUNK)], sem)
        for j in range(NCH)
    ]
    for c in copies:
        c.wait()
    pltpu.sync_copy(sm_v, out_hbm.at[pl.ds(wid * BPW, BPW)])


@functools.partial(
    pl.kernel,
    out_type=jax.ShapeDtypeStruct((B,), jnp.float32),
    mesh=_mesh,
    compiler_params=pltpu.CompilerParams(
        needs_layout_passes=False, use_tc_tiling_on_sc=False),
    scratch_types=[
        pltpu.VMEM((NCH, CHUNK), jnp.int32),
        pltpu.VMEM((BPW,), jnp.float32),
        pltpu.VMEM((BPW,), jnp.float32),
        pltpu.SemaphoreType.DMA,
    ],
)
def _sc_gather_user_add(users_hbm, su_hbm, smg_hbm, out_hbm,
                        uidx_v, su_v, sm_v, sem):
    wid = lax.axis_index("s") * NC + lax.axis_index("c")
    pltpu.sync_copy(users_hbm.at[pl.ds(wid * NCH, NCH)], uidx_v)
    pltpu.sync_copy(smg_hbm.at[pl.ds(wid * BPW, BPW)], sm_v)
    copies = [
        pltpu.async_copy(
            su_hbm.at[uidx_v.at[j]], su_v.at[pl.ds(j * CHUNK, CHUNK)], sem)
        for j in range(NCH)
    ]
    for c in copies:
        c.wait()
    for i in range(BPW // L):
        sl = pl.ds(i * L, L)
        su_v[sl] = su_v[sl] + sm_v[sl]
    pltpu.sync_copy(su_v, out_hbm.at[pl.ds(wid * BPW, BPW)])


def kernel(users, movies, user_table, movie_table, W, b):
    wf = W.reshape(-1).astype(jnp.float32)
    wu = wf[:D].reshape(D, 1)
    wm = wf[D:].reshape(D, 1)
    bias = b.astype(jnp.float32).reshape(1, 1)
    zero = jnp.zeros((1, 1), jnp.float32)
    sm = _matvec(movie_table.T, wm, zero)
    m2 = movies.astype(jnp.int32).reshape(NW * NCH, CHUNK)
    smg = _sc_gather_movie(m2, sm)
    su = _matvec(user_table.T, wu, bias)
    u2 = users.astype(jnp.int32).reshape(NW * NCH, CHUNK)
    out = _sc_gather_user_add(u2, su, smg)
    return out.reshape(B, 1)


# speedup vs baseline: 1.0807x; 1.0807x over previous
import functools

import jax
import jax.numpy as jnp
from jax import lax
from jax.experimental import pallas as pl
from jax.experimental.pallas import tpu as pltpu
from jax.experimental.pallas import tpu_sc as plsc

B = 16384
D = 32
NC = 2
NS = 16
L = 16
NW = NC * NS
BPW = B // NW
CHUNK = 128
NCH = BPW // CHUNK

BC = 65536


def _mv_body(x_ref, w_ref, b_ref, o_ref):
    o_ref[...] = jnp.sum(x_ref[...] * w_ref[...], axis=0) + b_ref[0, 0]


def _matvec(table_t, w, bias):
    n = table_t.shape[1]
    grid = pl.cdiv(n, BC)
    return pl.pallas_call(
        _mv_body,
        grid=(grid,),
        in_specs=[
            pl.BlockSpec((D, BC), lambda i: (0, i)),
            pl.BlockSpec((D, 1), lambda i: (0, 0)),
            pl.BlockSpec((1, 1), lambda i: (0, 0)),
        ],
        out_specs=pl.BlockSpec((BC,), lambda i: (i,)),
        out_shape=jax.ShapeDtypeStruct((n,), jnp.float32),
    )(table_t, w, bias)


_mesh = plsc.VectorSubcoreMesh(core_axis_name="c", subcore_axis_name="s")


@functools.partial(
    pl.kernel,
    out_type=jax.ShapeDtypeStruct((B,), jnp.float32),
    mesh=_mesh,
    compiler_params=pltpu.CompilerParams(
        needs_layout_passes=False, use_tc_tiling_on_sc=False),
    scratch_types=[
        pltpu.VMEM((NCH, CHUNK), jnp.int32),
        pltpu.VMEM((BPW,), jnp.float32),
        pltpu.SemaphoreType.DMA,
    ],
)
def _sc_gather_movie(movies_hbm, sm_hbm, out_hbm, midx_v, sm_v, sem):
    wid = lax.axis_index("s") * NC + lax.axis_index("c")
    pltpu.sync_copy(movies_hbm.at[pl.ds(wid * NCH, NCH)], midx_v)
    copies = [
        pltpu.async_copy(
            sm_hbm.at[midx_v.at[j]], sm_v.at[pl.ds(j * CHUNK, CHUNK)], sem)
        for j in range(NCH)
    ]
    for c in copies:
        c.wait()
    pltpu.sync_copy(sm_v, out_hbm.at[pl.ds(wid * BPW, BPW)])


@functools.partial(
    pl.kernel,
    out_type=jax.ShapeDtypeStruct((B,), jnp.float32),
    mesh=_mesh,
    compiler_params=pltpu.CompilerParams(
        needs_layout_passes=False, use_tc_tiling_on_sc=False),
    scratch_types=[
        pltpu.VMEM((NCH, CHUNK), jnp.int32),
        pltpu.VMEM((BPW,), jnp.float32),
        pltpu.VMEM((BPW,), jnp.float32),
        pltpu.SemaphoreType.DMA,
    ],
)
def _sc_gather_user_add(users_hbm, su_hbm, smg_hbm, out_hbm,
                        uidx_v, su_v, sm_v, sem):
    wid = lax.axis_index("s") * NC + lax.axis_index("c")
    pltpu.sync_copy(users_hbm.at[pl.ds(wid * NCH, NCH)], uidx_v)
    pltpu.sync_copy(smg_hbm.at[pl.ds(wid * BPW, BPW)], sm_v)
    copies = [
        pltpu.async_copy(
            su_hbm.at[uidx_v.at[j]], su_v.at[pl.ds(j * CHUNK, CHUNK)], sem)
        for j in range(NCH)
    ]
    for c in copies:
        c.wait()
    for i in range(BPW // L):
        sl = pl.ds(i * L, L)
        su_v[sl] = su_v[sl] + sm_v[sl]
    pltpu.sync_copy(su_v, out_hbm.at[pl.ds(wid * BPW, BPW)])


def kernel(users, movies, user_table, movie_table, W, b):
    wf = W.reshape(-1).astype(jnp.float32)
    wu = wf[:D].reshape(D, 1)
    wm = wf[D:].reshape(D, 1)
    bias = b.astype(jnp.float32).reshape(1, 1)
    zero = jnp.zeros((1, 1), jnp.float32)
    sm = _matvec(movie_table.T, wm, zero)
    m2 = movies.astype(jnp.int32).reshape(NW * NCH, CHUNK)
    smg = _sc_gather_movie(m2, sm)
    su = _matvec(user_table.T, wu, bias)
    u2 = users.astype(jnp.int32).reshape(NW * NCH, CHUNK)
    out = _sc_gather_user_add(u2, su, smg)
    return out.reshape(B, 1)
